# baseline (speedup 1.0000x reference)
import functools

import jax
import jax.numpy as jnp
import numpy as np
from jax import lax
from jax.experimental import pallas as pl
from jax.experimental.pallas import tpu as pltpu
from jax.experimental.pallas import tpu_sc as plsc

_BASE = np.array([[1, 0], [3, 0], [5, 0], [7, 0], [9, 0], [11, 0], [0, 2], [2, 2], [4, 2], [6, 2], [8, 2], [10, 2], [12, 2], [1, 4], [3, 4], [5, 4], [7, 4], [9, 4], [11, 4], [2, 6], [4, 6], [6, 6], [8, 6], [10, 6], [3, 8], [5, 8], [7, 8], [9, 8], [4, 10], [6, 10], [8, 10], [5, 12], [7, 12]], dtype=np.int64)
_H = 13
_W = 13
_OUT_HW = 49


def _build_taps():
    bx = _BASE[:, 0]
    by = _BASE[:, 1]
    bxm1 = np.maximum(bx - 1, 0)
    bxp1 = np.minimum(bx + 1, _H - 1)
    bym1 = np.maximum(by - 1, 0)
    byp1 = np.minimum(by + 1, _W - 1)
    mx = bx // 2 + (by // 2 + 1) % 2
    my = by // 2
    taps = [[] for _ in range(_OUT_HW)]
    for tx, ty in [(bx, by), (bxm1, by), (bxp1, by), (bx, byp1), (bx, bym1),
                   (bxm1, byp1), (bxm1, bym1)]:
        for i in range(len(bx)):
            taps[int(mx[i]) * 7 + int(my[i])].append(int(tx[i]) * _W + int(ty[i]))
    return taps


_TAPS = _build_taps()

_B = 1024
_C = 768
_B_TC = 512
_B_SC = _B - _B_TC
_N_WORKERS = 32
_PER_TEC = _B_SC * _C // _N_WORKERS
_S = 256
_ITERS = _PER_TEC // _S
_GROUPS = _S // 16


def _sc_pool_body(in_hbm, out_hbm, in_v, out_v, in_sem, out_sem):
    wid = lax.axis_index("s") * 2 + lax.axis_index("c")

    zeros = jnp.zeros((16,), jnp.float32)
    for p in range(2):
        for j in range(_OUT_HW):
            if not _TAPS[j]:
                for g in range(_GROUPS):
                    out_v[p, j, pl.ds(g * 16, 16)] = zeros

    def src_at(it):
        off = wid * _PER_TEC + it * _S
        b = _B_TC + off // _C
        c0 = off % _C
        return b, c0

    def dst_at(it):
        off = wid * _PER_TEC + it * _S
        return off // _C, off % _C

    b0, c00 = src_at(0)
    pltpu.async_copy(in_hbm.at[:, b0, pl.ds(c00, _S)], in_v.at[0],
                     in_sem.at[0])

    def it2_body(it2, carry):
        for sub in range(2):
            it = it2 * 2 + sub
            b, c0 = src_at(it)
            bo, co = dst_at(it)

            @pl.when(it + 1 < _ITERS)
            def _():
                bn, cn = src_at(it + 1)
                pltpu.async_copy(in_hbm.at[:, bn, pl.ds(cn, _S)],
                                 in_v.at[1 - sub], in_sem.at[1 - sub])

            @pl.when(it >= 2)
            def _():
                bp, cp = dst_at(it - 2)
                pltpu.make_async_copy(
                    out_v.at[sub],
                    out_hbm.at[:, bp, pl.ds(cp, _S)],
                    out_sem.at[sub]).wait()

            pltpu.make_async_copy(
                in_hbm.at[:, b, pl.ds(c0, _S)], in_v.at[sub],
                in_sem.at[sub]).wait()

            def grp_body(g, carry2):
                sl = pl.ds(g * 16, 16)
                vals = {}
                for my in range(7):
                    for mx in range(7):
                        j = mx * 7 + my
                        taps = _TAPS[j]
                        if not taps:
                            continue
                        for t in taps:
                            if t not in vals:
                                vals[t] = in_v[sub, t, sl]
                        acc = vals[taps[0]]
                        for t in taps[1:]:
                            acc = acc + vals[t]
                        out_v[sub, j, sl] = acc * jnp.float32(1.0 / 7.0)
                return carry2

            lax.fori_loop(0, _GROUPS, grp_body, 0)
            pltpu.async_copy(out_v.at[sub],
                             out_hbm.at[:, bo, pl.ds(co, _S)],
                             out_sem.at[sub])
        return carry

    lax.fori_loop(0, _ITERS // 2, it2_body, 0)

    for sub in range(2):
        it = _ITERS - 2 + sub
        bo, co = dst_at(it)
        pltpu.make_async_copy(
            out_v.at[sub], out_hbm.at[:, bo, pl.ds(co, _S)],
            out_sem.at[sub]).wait()


_sc_pool = functools.partial(
    pl.kernel,
    out_type=jax.ShapeDtypeStruct((_OUT_HW, _B_SC, _C), jnp.float32),
    mesh=plsc.VectorSubcoreMesh(core_axis_name="c", subcore_axis_name="s"),
    scratch_types=[
        pltpu.VMEM((2, _H * _W, _S), jnp.float32),
        pltpu.VMEM((2, _OUT_HW, _S), jnp.float32),
        pltpu.SemaphoreType.DMA((2,)),
        pltpu.SemaphoreType.DMA((2,)),
    ],
    compiler_params=pltpu.CompilerParams(needs_layout_passes=False),
)(_sc_pool_body)


def _tc_pool_body(x_ref, o_ref):
    zeros = jnp.zeros((8, 128), jnp.float32)
    vals = {}
    for my in range(7):
        for mx in range(7):
            j = mx * 7 + my
            taps = _TAPS[j]
            if not taps:
                o_ref[mx, my] = zeros
                continue
            for t in taps:
                if t not in vals:
                    vals[t] = x_ref[t // _W, t % _W]
            acc = vals[taps[0]]
            for t in taps[1:]:
                acc = acc + vals[t]
            o_ref[mx, my] = acc * jnp.float32(1.0 / 7.0)


_tc_pool = pl.pallas_call(
    _tc_pool_body,
    grid=(_B_TC // 8, _C // 128),
    in_specs=[pl.BlockSpec((_H, _W, 8, 128), lambda i, j: (0, 0, i, j))],
    out_specs=pl.BlockSpec((7, 7, 8, 128), lambda i, j: (0, 0, i, j)),
    out_shape=jax.ShapeDtypeStruct((7, 7, _B_TC, _C), jnp.float32),
)


@jax.jit
def kernel(input):
    b, ch, h, w = input.shape
    planes4 = input.transpose(2, 3, 0, 1)
    planes3 = planes4.reshape(h * w, b, ch)
    sc_out = _sc_pool(planes3).reshape(7, 7, _B_SC, ch)
    tc_out = _tc_pool(planes4)
    out = jnp.concatenate([tc_out, sc_out], axis=2)
    return out.transpose(2, 3, 0, 1)


# speedup vs baseline: 1.6257x; 1.6257x over previous
import functools

import jax
import jax.numpy as jnp
import numpy as np
from jax import lax
from jax.experimental import pallas as pl
from jax.experimental.pallas import tpu as pltpu
from jax.experimental.pallas import tpu_sc as plsc

_BASE = np.array([[1, 0], [3, 0], [5, 0], [7, 0], [9, 0], [11, 0], [0, 2], [2, 2], [4, 2], [6, 2], [8, 2], [10, 2], [12, 2], [1, 4], [3, 4], [5, 4], [7, 4], [9, 4], [11, 4], [2, 6], [4, 6], [6, 6], [8, 6], [10, 6], [3, 8], [5, 8], [7, 8], [9, 8], [4, 10], [6, 10], [8, 10], [5, 12], [7, 12]], dtype=np.int64)
_H = 13
_W = 13
_OUT_HW = 49


def _build_taps():
    bx = _BASE[:, 0]
    by = _BASE[:, 1]
    bxm1 = np.maximum(bx - 1, 0)
    bxp1 = np.minimum(bx + 1, _H - 1)
    bym1 = np.maximum(by - 1, 0)
    byp1 = np.minimum(by + 1, _W - 1)
    mx = bx // 2 + (by // 2 + 1) % 2
    my = by // 2
    taps = [[] for _ in range(_OUT_HW)]
    for tx, ty in [(bx, by), (bxm1, by), (bxp1, by), (bx, byp1), (bx, bym1),
                   (bxm1, byp1), (bxm1, bym1)]:
        for i in range(len(bx)):
            taps[int(mx[i]) * 7 + int(my[i])].append(int(tx[i]) * _W + int(ty[i]))
    return taps


_TAPS = _build_taps()

_B = 1024
_C = 768
_N_BC = _B * _C
_N_WORKERS = 32
_PER_TEC = _N_BC // _N_WORKERS
_S = 256
_ITERS = _PER_TEC // _S
_GROUPS = _S // 16


def _sc_pool_body(in_hbm, out_hbm, in_v, out_v, in_sem, out_sem):
    wid = lax.axis_index("s") * 2 + lax.axis_index("c")

    zeros = jnp.zeros((16,), jnp.float32)
    for p in range(2):
        for j in range(_OUT_HW):
            if not _TAPS[j]:
                for g in range(_GROUPS):
                    out_v[p, j, pl.ds(g * 16, 16)] = zeros

    def src_at(it):
        off = wid * _PER_TEC + it * _S
        b = off // _C
        c0 = off % _C
        return b, c0

    b0, c00 = src_at(0)
    pltpu.async_copy(in_hbm.at[:, b0, pl.ds(c00, _S)], in_v.at[0],
                     in_sem.at[0])

    def it2_body(it2, carry):
        for sub in range(2):
            it = it2 * 2 + sub
            b, c0 = src_at(it)

            @pl.when(it + 1 < _ITERS)
            def _():
                bn, cn = src_at(it + 1)
                pltpu.async_copy(in_hbm.at[:, bn, pl.ds(cn, _S)],
                                 in_v.at[1 - sub], in_sem.at[1 - sub])

            @pl.when(it >= 2)
            def _():
                bp, cp = src_at(it - 2)
                pltpu.make_async_copy(
                    out_v.at[sub],
                    out_hbm.at[:, bp, pl.ds(cp, _S)],
                    out_sem.at[sub]).wait()

            pltpu.make_async_copy(
                in_hbm.at[:, b, pl.ds(c0, _S)], in_v.at[sub],
                in_sem.at[sub]).wait()

            @plsc.parallel_loop(0, _GROUPS, 1)
            def grp_body(g):
                sl = pl.ds(g * 16, 16)
                vals = {}
                for my in range(7):
                    for mx in range(7):
                        j = mx * 7 + my
                        taps = _TAPS[j]
                        if not taps:
                            continue
                        for t in taps:
                            if t not in vals:
                                vals[t] = in_v[sub, t, sl]
                        acc = vals[taps[0]]
                        for t in taps[1:]:
                            acc = acc + vals[t]
                        out_v[sub, j, sl] = acc * jnp.float32(1.0 / 7.0)
            pltpu.async_copy(out_v.at[sub],
                             out_hbm.at[:, b, pl.ds(c0, _S)],
                             out_sem.at[sub])
        return carry

    lax.fori_loop(0, _ITERS // 2, it2_body, 0)

    for sub in range(2):
        it = _ITERS - 2 + sub
        b, c0 = src_at(it)
        pltpu.make_async_copy(
            out_v.at[sub], out_hbm.at[:, b, pl.ds(c0, _S)],
            out_sem.at[sub]).wait()


_sc_pool = functools.partial(
    pl.kernel,
    out_type=jax.ShapeDtypeStruct((_OUT_HW, _B, _C), jnp.float32),
    mesh=plsc.VectorSubcoreMesh(core_axis_name="c", subcore_axis_name="s"),
    scratch_types=[
        pltpu.VMEM((2, _H * _W, _S), jnp.float32),
        pltpu.VMEM((2, _OUT_HW, _S), jnp.float32),
        pltpu.SemaphoreType.DMA((2,)),
        pltpu.SemaphoreType.DMA((2,)),
    ],
    compiler_params=pltpu.CompilerParams(needs_layout_passes=False),
)(_sc_pool_body)


@jax.jit
def kernel(input):
    b, ch, h, w = input.shape
    planes = input.transpose(2, 3, 0, 1).reshape(h * w, b, ch)
    out = _sc_pool(planes)
    return out.reshape(7, 7, b, ch).transpose(2, 3, 0, 1)
